# Initial kernel scaffold; baseline (speedup 1.0000x reference)
import jax
import jax.numpy as jnp
from jax.experimental import pallas as pl


def kernel(image_tensor):
    raise NotImplementedError("write your pallas kernel here")


# speedup vs baseline: 39.7223x; 39.7223x over previous
import functools

import jax
import jax.numpy as jnp
import numpy as np
from jax.experimental import pallas as pl
from jax.experimental.pallas import tpu as pltpu

_H = _W = 128
_NPIX = _H * _W
_RING_BOUNDS = [(0, 16), (16, 32), (32, 48), (48, 64)]
_NRINGS = 4
_RB = 8
_CW = 2048


def _ring_masks():
    yy, xx = np.mgrid[0:_H, 0:_W]
    d2 = ((xx - 64) ** 2 + (yy - 64) ** 2).ravel()
    ms = []
    for r0, r1 in _RING_BOUNDS:
        ms.append(((d2 <= r1 * r1) & (d2 > r0 * r0)).astype(np.float32))
    return np.stack(ms)


_MASKS_NP = _ring_masks()
_RING_N = [int(m.sum()) for m in _MASKS_NP]
_RANK_A = [n // 2 for n in _RING_N]
_MASKS_B_NP = np.ascontiguousarray(
    np.broadcast_to(_MASKS_NP[:, None, :], (_NRINGS, _RB, _NPIX))
)


def _fold_lanes(y):
    while y.shape[1] > 128:
        h = y.shape[1] // 2
        y = y[:, :h] + y[:, h:]
    return y


def _body(x_ref, m_ref, out_ref, xs_ref):
    nchunks = _NPIX // _CW

    s1 = [jnp.zeros((_RB, 1), jnp.float32) for _ in range(_NRINGS)]
    s2 = [jnp.zeros((_RB, 1), jnp.float32) for _ in range(_NRINGS)]
    for c in range(nchunks):
        sl = slice(c * _CW, (c + 1) * _CW)
        x = jnp.floor(x_ref[:, sl] * 255.0)
        xs_ref[:, sl] = x
        xsq = x * x
        for r in range(_NRINGS):
            m = m_ref[r, :, sl]
            s1[r] += jnp.sum(_fold_lanes(x * m), axis=1, keepdims=True)
            s2[r] += jnp.sum(_fold_lanes(xsq * m), axis=1, keepdims=True)

    def bisect_step(_, carry):
        los, his = carry
        mids = [jnp.floor((los[r] + his[r]) * 0.5) for r in range(_NRINGS)]
        cnts = []
        for r in range(_NRINGS):
            acc = jnp.zeros((_RB, 128), jnp.float32)
            for c in range(nchunks):
                sl = slice(c * _CW, (c + 1) * _CW)
                x = xs_ref[:, sl]
                m = m_ref[r, :, sl]
                acc += _fold_lanes(jnp.where(x <= mids[r], m, 0.0))
            cnts.append(jnp.sum(acc, axis=1, keepdims=True))
        new_los, new_his = [], []
        for r in range(_NRINGS):
            ge = cnts[r] >= float(_RANK_A[r])
            new_his.append(jnp.where(ge, mids[r], his[r]))
            new_los.append(jnp.where(ge, los[r], mids[r] + 1.0))
        return new_los, new_his

    los = [jnp.zeros((_RB, 1), jnp.float32) for _ in range(_NRINGS)]
    his = [jnp.full((_RB, 1), 255.0, jnp.float32) for _ in range(_NRINGS)]
    los, his = jax.lax.fori_loop(0, 8, bisect_step, (los, his))
    mas = los

    meds = []
    for r in range(_NRINGS):
        acc_c = jnp.zeros((_RB, 128), jnp.float32)
        acc_m = jnp.full((_RB, 128), 1e9, jnp.float32)
        for c in range(nchunks):
            sl = slice(c * _CW, (c + 1) * _CW)
            x = xs_ref[:, sl]
            m = m_ref[r, :, sl]
            le = x <= mas[r]
            acc_c += _fold_lanes(jnp.where(le, m, 0.0))
            above = jnp.where((~le) & (m > 0.5), x, 1e9)
            while above.shape[1] > 128:
                h = above.shape[1] // 2
                above = jnp.minimum(above[:, :h], above[:, h:])
            acc_m = jnp.minimum(acc_m, above)
        cnt_a = jnp.sum(acc_c, axis=1, keepdims=True)
        min_above = jnp.min(acc_m, axis=1, keepdims=True)
        mb = jnp.where(cnt_a >= float(_RANK_A[r] + 1), mas[r], min_above)
        meds.append(0.5 * (mas[r] + mb))

    for r in range(_NRINGS):
        inv_n = 1.0 / float(_RING_N[r])
        mean = s1[r] * inv_n
        var = s2[r] * inv_n - mean * mean
        std = jnp.sqrt(jnp.maximum(var, 0.0))
        out_ref[:, 3 * r : 3 * r + 1] = mean
        out_ref[:, 3 * r + 1 : 3 * r + 2] = std
        out_ref[:, 3 * r + 2 : 3 * r + 3] = meds[r]


@jax.jit
def kernel(image_tensor):
    b = image_tensor.shape[0]
    rows = b * 3
    x2 = image_tensor.reshape(rows, _NPIX)
    masks = jnp.asarray(_MASKS_B_NP)
    grid = (rows // _RB,)
    out = pl.pallas_call(
        _body,
        grid=grid,
        in_specs=[
            pl.BlockSpec((_RB, _NPIX), lambda i: (i, 0)),
            pl.BlockSpec((_NRINGS, _RB, _NPIX), lambda i: (0, 0, 0)),
        ],
        out_specs=pl.BlockSpec((_RB, 12), lambda i: (i, 0)),
        out_shape=jax.ShapeDtypeStruct((rows, 12), jnp.float32),
        scratch_shapes=[pltpu.VMEM((_RB, _NPIX), jnp.float32)],
        compiler_params=pltpu.CompilerParams(
            dimension_semantics=("parallel",),
        ),
    )(x2, masks)
    return (
        out.reshape(b, 3, _NRINGS, 3)
        .transpose(0, 2, 3, 1)
        .reshape(b, _NRINGS, 9)
    )
